# baseline (speedup 1.0000x reference)
import functools

import jax
import jax.numpy as jnp
from jax import lax
from jax.experimental import pallas as pl
from jax.experimental.pallas import tpu as pltpu
from jax.experimental.pallas import tpu_sc as plsc

N_ENSEMBLE = 10
VOCAB = 100000
DIM = 64
B, L = 4096, 50
PK = DIM // 2

_info = plsc.get_sparse_core_info()
NC, NS = _info.num_cores, _info.num_subcores
NW = NC * NS
BW = B // NW

_mesh = plsc.VectorSubcoreMesh(core_axis_name="c", subcore_axis_name="s")


@functools.partial(
    pl.kernel,
    mesh=_mesh,
    compiler_params=pltpu.CompilerParams(
        needs_layout_passes=False, use_tc_tiling_on_sc=False
    ),
    out_type=jax.ShapeDtypeStruct((L, DIM // 8, NW, 8, BW), jnp.float32),
    scratch_types=(
        [pltpu.VMEM((L, BW), jnp.int32)]
        + [pltpu.VMEM((BW, PK), jnp.int32) for _ in range(2)]
        + [pltpu.VMEM((DIM // 8, 8, BW), jnp.float32) for _ in range(2)]
        + [pltpu.SemaphoreType.DMA for _ in range(4)]
    ),
)
def _table_lookup(idx_hbm, tab_hbm, out, idx_v, rows0, rows1, tr0, tr1,
                  g0, g1, w0, w1):
    rows = (rows0, rows1)
    trs = (tr0, tr1)
    gsem = (g0, g1)
    wsem = (w0, w1)
    wid = lax.axis_index("s") * NC + lax.axis_index("c")
    pltpu.sync_copy(idx_hbm.at[:, wid, :], idx_v)

    iota = lax.iota(jnp.int32, 16)
    row_ids = [g * 16 + iota for g in range(8)]

    def start_g(l, p):
        pltpu.async_copy(tab_hbm.at[idx_v.at[l]], rows[p], gsem[p])

    def wait_g(p):
        pltpu.make_async_copy(tab_hbm.at[idx_v.at[0]], rows[p], gsem[p]).wait()

    def start_s(l, p):
        pltpu.async_copy(trs[p], out.at[l, :, wid, :, :], wsem[p])

    def wait_s(p):
        pltpu.make_async_copy(trs[p], out.at[0, :, wid, :, :], wsem[p]).wait()

    def transpose(p):
        def gbody(dg, _):
            for pp in range(4):
                col = jnp.zeros((16,), jnp.int32) + (dg * 4 + pp)
                for g in range(8):
                    v = plsc.load_gather(rows[p], [row_ids[g], col])
                    vb = plsc.bitcast(v, jnp.bfloat16)
                    lo, hi = plsc.unpack(vb, format=plsc.PackFormat.INTERLEAVED)
                    trs[p][dg, 2 * pp, pl.ds(g * 16, 16)] = lo
                    trs[p][dg, 2 * pp + 1, pl.ds(g * 16, 16)] = hi
            return 0

        lax.fori_loop(0, DIM // 8, gbody, 0)

    start_g(0, 0)
    for l in (0, 1):
        wait_g(l % 2)
        start_g(l + 1, (l + 1) % 2)
        transpose(l % 2)
        start_s(l, l % 2)

    def pair_body(m, _):
        for k in (0, 1):
            l = 2 * m + k
            wait_g(k)
            start_g(l + 1, 1 - k)
            wait_s(k)
            transpose(k)
            start_s(l, k)
        return 0

    lax.fori_loop(1, (L - 2) // 2, pair_body, 0)

    for l in (L - 2, L - 1):
        p = l % 2
        wait_g(p)
        if l + 1 < L:
            start_g(l + 1, 1 - p)
        wait_s(p)
        transpose(p)
        start_s(l, p)
    wait_s(0)
    wait_s(1)


def kernel(indices, tables):
    idx3 = jnp.transpose(indices.astype(jnp.int32)).reshape(L, NW, BW)
    outs = []
    for t in range(N_ENSEMBLE):
        xb = (tables[t] * 8.0).astype(jnp.bfloat16)
        u = lax.bitcast_convert_type(xb, jnp.uint16)
        tab32 = (u[:, 0::2].astype(jnp.uint32)
                 | (u[:, 1::2].astype(jnp.uint32) << 16))
        tab32 = lax.bitcast_convert_type(tab32, jnp.int32)
        raw = _table_lookup(idx3, tab32)
        out = raw.transpose(2, 4, 0, 1, 3).reshape(B, L, DIM)
        outs.append(out)
    return tuple(outs)


# speedup vs baseline: 2.7678x; 2.7678x over previous
import functools

import jax
import jax.numpy as jnp
from jax import lax
from jax.experimental import pallas as pl
from jax.experimental.pallas import tpu as pltpu
from jax.experimental.pallas import tpu_sc as plsc

N_ENSEMBLE = 10
VOCAB = 100000
DIM = 64
B, L = 4096, 50
PK = DIM // 2

_info = plsc.get_sparse_core_info()
NC, NS = _info.num_cores, _info.num_subcores
NW = NC * NS
BW = B // NW

_mesh = plsc.VectorSubcoreMesh(core_axis_name="c", subcore_axis_name="s")


@functools.partial(
    pl.kernel,
    mesh=_mesh,
    compiler_params=pltpu.CompilerParams(
        needs_layout_passes=False, use_tc_tiling_on_sc=False
    ),
    out_type=jax.ShapeDtypeStruct((L, DIM // 8, NW, 8, BW), jnp.float32),
    scratch_types=(
        [pltpu.VMEM((L, BW), jnp.int32)]
        + [pltpu.VMEM((BW, PK), jnp.int32) for _ in range(2)]
        + [pltpu.VMEM((DIM // 8, 8, BW), jnp.float32) for _ in range(2)]
        + [pltpu.SemaphoreType.DMA for _ in range(4)]
    ),
)
def _table_lookup(idx_hbm, tab_hbm, out, idx_v, rows0, rows1, tr0, tr1,
                  g0, g1, w0, w1):
    rows = (rows0, rows1)
    trs = (tr0, tr1)
    gsem = (g0, g1)
    wsem = (w0, w1)
    wid = lax.axis_index("s") * NC + lax.axis_index("c")
    pltpu.sync_copy(idx_hbm.at[:, wid, :], idx_v)

    iota = lax.iota(jnp.int32, 16)
    row_ids = [g * 16 + iota for g in range(8)]

    def start_g(l, p):
        pltpu.async_copy(tab_hbm.at[idx_v.at[l]], rows[p], gsem[p])

    def wait_g(p):
        pltpu.make_async_copy(tab_hbm.at[idx_v.at[0]], rows[p], gsem[p]).wait()

    def start_s(l, p):
        pltpu.async_copy(trs[p], out.at[l, :, wid, :, :], wsem[p])

    def wait_s(p):
        pltpu.make_async_copy(trs[p], out.at[0, :, wid, :, :], wsem[p]).wait()

    def transpose(p):
        def gbody(dg, _):
            for pp in range(4):
                col = jnp.zeros((16,), jnp.int32) + (dg * 4 + pp)
                for g in range(8):
                    v = plsc.load_gather(rows[p], [row_ids[g], col])
                    lo = plsc.bitcast(v << 16, jnp.float32)
                    hi = plsc.bitcast(v & jnp.int32(-65536), jnp.float32)
                    trs[p][dg, 2 * pp, pl.ds(g * 16, 16)] = lo
                    trs[p][dg, 2 * pp + 1, pl.ds(g * 16, 16)] = hi
            return 0

        lax.fori_loop(0, DIM // 8, gbody, 0)

    start_g(0, 0)
    for l in (0, 1):
        wait_g(l % 2)
        start_g(l + 1, (l + 1) % 2)
        transpose(l % 2)
        start_s(l, l % 2)

    def pair_body(m, _):
        for k in (0, 1):
            l = 2 * m + k
            wait_g(k)
            start_g(l + 1, 1 - k)
            wait_s(k)
            transpose(k)
            start_s(l, k)
        return 0

    lax.fori_loop(1, (L - 2) // 2, pair_body, 0)

    for l in (L - 2, L - 1):
        p = l % 2
        wait_g(p)
        if l + 1 < L:
            start_g(l + 1, 1 - p)
        wait_s(p)
        transpose(p)
        start_s(l, p)
    wait_s(0)
    wait_s(1)


def kernel(indices, tables):
    idx3 = jnp.transpose(indices.astype(jnp.int32)).reshape(L, NW, BW)
    outs = []
    for t in range(N_ENSEMBLE):
        xb = (tables[t] * 8.0).astype(jnp.bfloat16).reshape(VOCAB, PK, 2)
        tab32 = lax.bitcast_convert_type(xb, jnp.int32)
        raw = _table_lookup(idx3, tab32)
        out = raw.transpose(2, 4, 0, 1, 3).reshape(B, L, DIM)
        outs.append(out)
    return tuple(outs)


# speedup vs baseline: 6.4875x; 2.3439x over previous
import functools

import jax
import jax.numpy as jnp
from jax import lax
from jax.experimental import pallas as pl
from jax.experimental.pallas import tpu as pltpu
from jax.experimental.pallas import tpu_sc as plsc

N_ENSEMBLE = 10
VOCAB = 100000
DIM = 64
B, L = 4096, 50
PK = DIM // 2

_info = plsc.get_sparse_core_info()
NC, NS = _info.num_cores, _info.num_subcores
NW = NC * NS
BW = B // NW

_mesh = plsc.VectorSubcoreMesh(core_axis_name="c", subcore_axis_name="s")


@functools.partial(
    pl.kernel,
    mesh=_mesh,
    compiler_params=pltpu.CompilerParams(
        needs_layout_passes=False, use_tc_tiling_on_sc=False
    ),
    out_type=jax.ShapeDtypeStruct((L, DIM // 8, NW, 8, BW), jnp.float32),
    scratch_types=(
        [pltpu.VMEM((L, BW), jnp.int32)]
        + [pltpu.VMEM((BW, PK), jnp.int32) for _ in range(2)]
        + [pltpu.VMEM((DIM // 8, 8, BW), jnp.float32) for _ in range(2)]
        + [pltpu.SemaphoreType.DMA for _ in range(4)]
    ),
)
def _table_lookup(idx_hbm, tab_hbm, out, idx_v, rows0, rows1, tr0, tr1,
                  g0, g1, w0, w1):
    rows = (rows0, rows1)
    trs = (tr0, tr1)
    gsem = (g0, g1)
    wsem = (w0, w1)
    wid = lax.axis_index("s") * NC + lax.axis_index("c")
    pltpu.sync_copy(idx_hbm.at[:, wid, :], idx_v)

    iota = lax.iota(jnp.int32, 16)
    row_ids = [g * 16 + iota for g in range(8)]

    def start_g(l, p):
        pltpu.async_copy(tab_hbm.at[idx_v.at[l]], rows[p], gsem[p])

    def wait_g(p):
        pltpu.make_async_copy(tab_hbm.at[idx_v.at[0]], rows[p], gsem[p]).wait()

    def start_s(l, p):
        pltpu.async_copy(trs[p], out.at[l, :, wid, :, :], wsem[p])

    def wait_s(p):
        pltpu.make_async_copy(trs[p], out.at[0, :, wid, :, :], wsem[p]).wait()

    def transpose(p):
        @functools.partial(plsc.parallel_loop, 0, PK // 8)
        def _(kg):
            for kk in range(8):
                col = jnp.zeros((16,), jnp.int32) + (kg * 8 + kk)
                for g in range(8):
                    v = plsc.load_gather(rows[p], [row_ids[g], col])
                    lo = plsc.bitcast(v << 16, jnp.float32)
                    hi = plsc.bitcast(v & jnp.int32(-65536), jnp.float32)
                    trs[p][kg, kk, pl.ds(g * 16, 16)] = lo
                    trs[p][kg + PK // 8, kk, pl.ds(g * 16, 16)] = hi

    start_g(0, 0)
    for l in (0, 1):
        wait_g(l % 2)
        start_g(l + 1, (l + 1) % 2)
        transpose(l % 2)
        start_s(l, l % 2)

    def pair_body(m, _):
        for k in (0, 1):
            l = 2 * m + k
            wait_g(k)
            start_g(l + 1, 1 - k)
            wait_s(k)
            transpose(k)
            start_s(l, k)
        return 0

    lax.fori_loop(1, (L - 2) // 2, pair_body, 0)

    for l in (L - 2, L - 1):
        p = l % 2
        wait_g(p)
        if l + 1 < L:
            start_g(l + 1, 1 - p)
        wait_s(p)
        transpose(p)
        start_s(l, p)
    wait_s(0)
    wait_s(1)


def kernel(indices, tables):
    idx3 = jnp.transpose(indices.astype(jnp.int32)).reshape(L, NW, BW)
    outs = []
    for t in range(N_ENSEMBLE):
        xb = (tables[t] * 8.0).astype(jnp.bfloat16)
        u = lax.bitcast_convert_type(xb, jnp.uint16)
        tab32 = lax.bitcast_convert_type(
            u[:, :PK].astype(jnp.uint32) | (u[:, PK:].astype(jnp.uint32) << 16),
            jnp.int32)
        raw = _table_lookup(idx3, tab32)
        out = raw.transpose(2, 4, 0, 1, 3).reshape(B, L, DIM)
        outs.append(out)
    return tuple(outs)
